# baseline (speedup 1.0000x reference)
import functools

import jax
import jax.numpy as jnp
from jax.experimental import pallas as pl
from jax.experimental.pallas import tpu as pltpu

DIM = 2048
NH = 16
DOWN = 512
UP = 2048
RHD = 32
VHD = 128
HID = 1408
NE = 8
TOPK = 2
EPS = 1e-5
S = 2048

QHD = UP // NH

BQ = 512
BK = 512
BM = 128
NHB = 2
BH = HID // NHB
P = S * TOPK + NE * BM
NBLK = P // BM
NTB = S // BM


def _rmsnorm(h, w):
    return h * jax.lax.rsqrt(jnp.mean(h * h, axis=-1, keepdims=True) + EPS) * w


def _rope(t, cs):
    t2 = t.reshape(t.shape[:-1] + (-1, 2))
    c = cs[:, None, :, 0]
    s = cs[:, None, :, 1]
    o0 = t2[..., 0] * c - t2[..., 1] * s
    o1 = t2[..., 0] * s + t2[..., 1] * c
    return jnp.stack([o0, o1], axis=-1).reshape(t.shape)


def _flash_kernel(qc_ref, qr_ref, kc_ref, kr_ref, v_ref, o_ref):
    qi = pl.program_id(1)
    scale = 1.0 / jnp.sqrt(jnp.float32(QHD + RHD))
    qc = qc_ref[...] * scale
    qr = qr_ref[0] * scale

    def scores(j):
        kc = kc_ref[pl.ds(j * BK, BK), :]
        kr = kr_ref[pl.ds(j * BK, BK), :]
        s = jax.lax.dot_general(qc, kc, (((1,), (1,)), ((), ())),
                                preferred_element_type=jnp.float32)
        s += jax.lax.dot_general(qr, kr, (((1,), (1,)), ((), ())),
                                 preferred_element_type=jnp.float32)
        return s

    def update(j, s, carry):
        acc, m, l = carry
        v = v_ref[pl.ds(j * BK, BK), :]
        m_new = jnp.maximum(m, jnp.max(s, axis=-1, keepdims=True))
        p = jnp.exp(s - m_new)
        alpha = jnp.exp(m - m_new)
        l_new = l * alpha + jnp.sum(p, axis=-1, keepdims=True)
        acc_new = acc * alpha + jax.lax.dot_general(
            p, v, (((1,), (0,)), ((), ())), preferred_element_type=jnp.float32)
        return acc_new, m_new, l_new

    def body(j, carry):
        return update(j, scores(j), carry)

    acc = jnp.zeros((BQ, VHD), jnp.float32)
    m0 = jnp.full((BQ, 1), -jnp.inf, jnp.float32)
    l0 = jnp.zeros((BQ, 1), jnp.float32)
    carry = jax.lax.fori_loop(0, qi, body, (acc, m0, l0))
    s = scores(qi)
    mask = (jax.lax.broadcasted_iota(jnp.int32, (BQ, BK), 0)
            >= jax.lax.broadcasted_iota(jnp.int32, (BQ, BK), 1))
    s = jnp.where(mask, s, -1e30)
    acc, m, l = update(qi, s, carry)
    o_ref[...] = acc / l


def _flash_attn(q_c, q_r, k_c, k_r, v):
    return pl.pallas_call(
        _flash_kernel,
        grid=(NH, S // BQ),
        in_specs=[
            pl.BlockSpec((BQ, QHD), lambda h, qi: (qi, h)),
            pl.BlockSpec((1, BQ, RHD), lambda h, qi: (h, qi, 0)),
            pl.BlockSpec((S, QHD), lambda h, qi: (0, h)),
            pl.BlockSpec((S, RHD), lambda h, qi: (0, 0)),
            pl.BlockSpec((S, VHD), lambda h, qi: (0, h)),
        ],
        out_specs=pl.BlockSpec((BQ, VHD), lambda h, qi: (qi, h)),
        out_shape=jax.ShapeDtypeStruct((S, NH * VHD), jnp.float32),
    )(q_c, q_r, k_c, k_r, v)


def _ffn_up_kernel(be_ref, fs_ref, w1_ref, w3_ref, g_ref):
    fs = fs_ref[...].astype(jnp.bfloat16)
    h1 = jax.lax.dot_general(fs, w1_ref[0], (((1,), (1,)), ((), ())),
                             preferred_element_type=jnp.float32)
    h3 = jax.lax.dot_general(fs, w3_ref[0], (((1,), (1,)), ((), ())),
                             preferred_element_type=jnp.float32)
    g = h1 * jax.lax.logistic(h1) * h3
    g_ref[...] = g.astype(jnp.bfloat16)


def _ffn_down_kernel(be_ref, g_ref, w2_ref, o_ref):
    o_ref[...] = jax.lax.dot_general(g_ref[...], w2_ref[0],
                                     (((1,), (1,)), ((), ())),
                                     preferred_element_type=jnp.float32)


def _grouped_ffn(be, fs, w1, w3, w2, nblk):
    g = pl.pallas_call(
        _ffn_up_kernel,
        grid_spec=pltpu.PrefetchScalarGridSpec(
            num_scalar_prefetch=1,
            grid=(nblk,),
            in_specs=[
                pl.BlockSpec((BM, DIM), lambda i, be: (i, 0)),
                pl.BlockSpec((1, HID, DIM), lambda i, be: (be[i], 0, 0)),
                pl.BlockSpec((1, HID, DIM), lambda i, be: (be[i], 0, 0)),
            ],
            out_specs=pl.BlockSpec((BM, HID), lambda i, be: (i, 0)),
        ),
        out_shape=jax.ShapeDtypeStruct((nblk * BM, HID), jnp.bfloat16),
    )(be, fs, w1, w3)
    return pl.pallas_call(
        _ffn_down_kernel,
        grid_spec=pltpu.PrefetchScalarGridSpec(
            num_scalar_prefetch=1,
            grid=(nblk,),
            in_specs=[
                pl.BlockSpec((BM, HID), lambda i, be: (i, 0)),
                pl.BlockSpec((1, DIM, HID), lambda i, be: (be[i], 0, 0)),
            ],
            out_specs=pl.BlockSpec((BM, DIM), lambda i, be: (i, 0)),
        ),
        out_shape=jax.ShapeDtypeStruct((nblk * BM, DIM), jnp.float32),
    )(be, g, w2)


def kernel(x, pos_cis, attn_norm_w, Wdkv, bdkv, Wuk, buk, Wuv, buv, Wdq, bdq,
           Wuq, buq, Wqr, bqr, Wkr, bkr, Wo, bo, ffn_norm_w, gate_w,
           ew1, ew2, ew3, sw1, sw2, sw3):
    b, s, _ = x.shape
    xf = x.reshape(s, DIM)
    h = _rmsnorm(xf, attn_norm_w)
    c_kv = h @ Wdkv.T + bdkv
    c_q = h @ Wdq.T + bdq
    k_c = c_kv @ Wuk.T + buk
    v = c_kv @ Wuv.T + buv
    q_c = c_q @ Wuq.T + buq
    q_pe = (c_q @ Wqr.T + bqr).reshape(s, NH, RHD)
    k_pe = (h @ Wkr.T + bkr).reshape(s, 1, RHD)
    q_r = _rope(q_pe, pos_cis).transpose(1, 0, 2)
    k_r = _rope(k_pe, pos_cis).reshape(s, RHD)

    o = _flash_attn(q_c, q_r, k_c, k_r, v)
    h_att = xf + o @ Wo.T + bo

    f = _rmsnorm(h_att, ffn_norm_w)
    logits = f @ gate_w.T
    sc = jax.nn.softmax(logits, axis=-1)
    tw, ti = jax.lax.top_k(sc, TOPK)

    e_flat = ti.reshape(-1).astype(jnp.int32)
    order = jnp.argsort(e_flat, stable=True).astype(jnp.int32)
    sorted_e = e_flat[order]
    counts = jnp.sum(jax.nn.one_hot(e_flat, NE, dtype=jnp.int32), axis=0)
    pc = ((counts + BM - 1) // BM) * BM
    group_start = jnp.cumsum(counts) - counts
    padded_start = jnp.cumsum(pc) - pc
    r_in_group = jnp.arange(S * TOPK, dtype=jnp.int32) - group_start[sorted_e]
    dest = padded_start[sorted_e] + r_in_group
    src = jnp.zeros((P,), jnp.int32).at[dest].set(order)
    tok = src // TOPK
    blk_start = padded_start // BM
    bids = jnp.arange(NBLK, dtype=jnp.int32)
    blk_expert = (jnp.sum(bids[:, None] >= blk_start[None, :], axis=1)
                  .astype(jnp.int32) - 1)

    fs = jnp.take(f, tok, axis=0)
    bf = jnp.bfloat16
    ye = _grouped_ffn(blk_expert, fs, ew1.astype(bf), ew3.astype(bf),
                      ew2.astype(bf), NBLK)
    y_sh = _grouped_ffn(jnp.zeros((NTB,), jnp.int32), f,
                        sw1[None].astype(bf), sw3[None].astype(bf),
                        sw2[None].astype(bf), NTB)

    pos_flat = jnp.zeros((S * TOPK,), jnp.int32).at[order].set(dest)
    pos = pos_flat.reshape(S, TOPK)
    y = (tw[:, 0:1] * jnp.take(ye, pos[:, 0], axis=0)
         + tw[:, 1:2] * jnp.take(ye, pos[:, 1], axis=0)
         + y_sh)
    return (h_att + y).reshape(b, s, DIM)


# speedup vs baseline: 1.0497x; 1.0497x over previous
import functools

import jax
import jax.numpy as jnp
from jax.experimental import pallas as pl
from jax.experimental.pallas import tpu as pltpu

DIM = 2048
NH = 16
DOWN = 512
UP = 2048
RHD = 32
VHD = 128
HID = 1408
NE = 8
TOPK = 2
EPS = 1e-5
S = 2048

QHD = UP // NH

BQ = 512
BK = 512
BM = 128
NHB = 2
BH = HID // NHB
P = S * TOPK + NE * BM
NBLK = P // BM
NTB = S // BM


def _rmsnorm(h, w):
    return h * jax.lax.rsqrt(jnp.mean(h * h, axis=-1, keepdims=True) + EPS) * w


def _rope(t, cs):
    t2 = t.reshape(t.shape[:-1] + (-1, 2))
    c = cs[:, None, :, 0]
    s = cs[:, None, :, 1]
    o0 = t2[..., 0] * c - t2[..., 1] * s
    o1 = t2[..., 0] * s + t2[..., 1] * c
    return jnp.stack([o0, o1], axis=-1).reshape(t.shape)


def _flash_kernel(qc_ref, qr_ref, kc_ref, kr_ref, v_ref, o_ref):
    qi = pl.program_id(1)
    scale = 1.0 / jnp.sqrt(jnp.float32(QHD + RHD))
    qc = qc_ref[...] * scale
    qr = qr_ref[0] * scale

    def scores(j):
        kc = kc_ref[pl.ds(j * BK, BK), :]
        kr = kr_ref[pl.ds(j * BK, BK), :]
        s = jax.lax.dot_general(qc, kc, (((1,), (1,)), ((), ())),
                                preferred_element_type=jnp.float32)
        s += jax.lax.dot_general(qr, kr, (((1,), (1,)), ((), ())),
                                 preferred_element_type=jnp.float32)
        return s

    def update(j, s, carry):
        acc, m, l = carry
        v = v_ref[pl.ds(j * BK, BK), :]
        m_new = jnp.maximum(m, jnp.max(s, axis=-1, keepdims=True))
        p = jnp.exp(s - m_new)
        alpha = jnp.exp(m - m_new)
        l_new = l * alpha + jnp.sum(p, axis=-1, keepdims=True)
        acc_new = acc * alpha + jax.lax.dot_general(
            p, v, (((1,), (0,)), ((), ())), preferred_element_type=jnp.float32)
        return acc_new, m_new, l_new

    def body(j, carry):
        return update(j, scores(j), carry)

    acc = jnp.zeros((BQ, VHD), jnp.float32)
    m0 = jnp.full((BQ, 1), -jnp.inf, jnp.float32)
    l0 = jnp.zeros((BQ, 1), jnp.float32)
    carry = jax.lax.fori_loop(0, qi, body, (acc, m0, l0))
    s = scores(qi)
    mask = (jax.lax.broadcasted_iota(jnp.int32, (BQ, BK), 0)
            >= jax.lax.broadcasted_iota(jnp.int32, (BQ, BK), 1))
    s = jnp.where(mask, s, -1e30)
    acc, m, l = update(qi, s, carry)
    o_ref[...] = acc / l


def _flash_attn(q_c, q_r, k_c, k_r, v):
    return pl.pallas_call(
        _flash_kernel,
        grid=(NH, S // BQ),
        in_specs=[
            pl.BlockSpec((BQ, QHD), lambda h, qi: (qi, h)),
            pl.BlockSpec((1, BQ, RHD), lambda h, qi: (h, qi, 0)),
            pl.BlockSpec((S, QHD), lambda h, qi: (0, h)),
            pl.BlockSpec((S, RHD), lambda h, qi: (0, 0)),
            pl.BlockSpec((S, VHD), lambda h, qi: (0, h)),
        ],
        out_specs=pl.BlockSpec((BQ, VHD), lambda h, qi: (qi, h)),
        out_shape=jax.ShapeDtypeStruct((S, NH * VHD), jnp.float32),
    )(q_c, q_r, k_c, k_r, v)


def _ffn_up_kernel(be_ref, fs_ref, w1_ref, w3_ref, g_ref):
    fs = fs_ref[...]
    h1 = jax.lax.dot_general(fs, w1_ref[0], (((1,), (1,)), ((), ())),
                             preferred_element_type=jnp.float32)
    h3 = jax.lax.dot_general(fs, w3_ref[0], (((1,), (1,)), ((), ())),
                             preferred_element_type=jnp.float32)
    g_ref[...] = h1 * jax.lax.logistic(h1) * h3


def _ffn_down_kernel(be_ref, g_ref, w2_ref, o_ref):
    o_ref[...] = jax.lax.dot_general(g_ref[...], w2_ref[0],
                                     (((1,), (1,)), ((), ())),
                                     preferred_element_type=jnp.float32)


def _grouped_ffn(be, fs, w1, w3, w2, nblk):
    g = pl.pallas_call(
        _ffn_up_kernel,
        grid_spec=pltpu.PrefetchScalarGridSpec(
            num_scalar_prefetch=1,
            grid=(nblk,),
            in_specs=[
                pl.BlockSpec((BM, DIM), lambda i, be: (i, 0)),
                pl.BlockSpec((1, HID, DIM), lambda i, be: (be[i], 0, 0)),
                pl.BlockSpec((1, HID, DIM), lambda i, be: (be[i], 0, 0)),
            ],
            out_specs=pl.BlockSpec((BM, HID), lambda i, be: (i, 0)),
        ),
        out_shape=jax.ShapeDtypeStruct((nblk * BM, HID), jnp.float32),
    )(be, fs, w1, w3)
    return pl.pallas_call(
        _ffn_down_kernel,
        grid_spec=pltpu.PrefetchScalarGridSpec(
            num_scalar_prefetch=1,
            grid=(nblk,),
            in_specs=[
                pl.BlockSpec((BM, HID), lambda i, be: (i, 0)),
                pl.BlockSpec((1, DIM, HID), lambda i, be: (be[i], 0, 0)),
            ],
            out_specs=pl.BlockSpec((BM, DIM), lambda i, be: (i, 0)),
        ),
        out_shape=jax.ShapeDtypeStruct((nblk * BM, DIM), jnp.float32),
    )(be, g, w2)


def kernel(x, pos_cis, attn_norm_w, Wdkv, bdkv, Wuk, buk, Wuv, buv, Wdq, bdq,
           Wuq, buq, Wqr, bqr, Wkr, bkr, Wo, bo, ffn_norm_w, gate_w,
           ew1, ew2, ew3, sw1, sw2, sw3):
    b, s, _ = x.shape
    xf = x.reshape(s, DIM)
    h = _rmsnorm(xf, attn_norm_w)
    c_kv = h @ Wdkv.T + bdkv
    c_q = h @ Wdq.T + bdq
    k_c = c_kv @ Wuk.T + buk
    v = c_kv @ Wuv.T + buv
    q_c = c_q @ Wuq.T + buq
    q_pe = (c_q @ Wqr.T + bqr).reshape(s, NH, RHD)
    k_pe = (h @ Wkr.T + bkr).reshape(s, 1, RHD)
    q_r = _rope(q_pe, pos_cis).transpose(1, 0, 2)
    k_r = _rope(k_pe, pos_cis).reshape(s, RHD)

    o = _flash_attn(q_c, q_r, k_c, k_r, v)
    h_att = xf + o @ Wo.T + bo

    f = _rmsnorm(h_att, ffn_norm_w)
    logits = f @ gate_w.T
    sc = jax.nn.softmax(logits, axis=-1)
    lanes = jnp.arange(NE, dtype=jnp.int32)[None, :]
    i1 = jnp.argmax(sc, axis=-1).astype(jnp.int32)
    w1v = jnp.max(sc, axis=-1)
    sc2 = jnp.where(lanes == i1[:, None], -jnp.inf, sc)
    i2 = jnp.argmax(sc2, axis=-1).astype(jnp.int32)
    w2v = jnp.max(sc2, axis=-1)
    tw = jnp.stack([w1v, w2v], axis=-1)
    ti = jnp.stack([i1, i2], axis=-1)

    e_flat = ti.reshape(-1)
    oh = (e_flat[:, None] == lanes).astype(jnp.int32)
    ranks = jnp.cumsum(oh, axis=0) - oh
    rank = jnp.take_along_axis(ranks, e_flat[:, None], axis=1)[:, 0]
    counts = ranks[-1] + oh[-1]
    pc = ((counts + BM - 1) // BM) * BM
    padded_start = jnp.cumsum(pc) - pc
    dest = padded_start[e_flat] + rank
    src = jnp.zeros((P,), jnp.int32).at[dest].set(
        jnp.arange(S * TOPK, dtype=jnp.int32))
    tok = src // TOPK
    blk_start = padded_start // BM
    bids = jnp.arange(NBLK, dtype=jnp.int32)
    blk_expert = (jnp.sum(bids[:, None] >= blk_start[None, :], axis=1)
                  .astype(jnp.int32) - 1)

    fs = jnp.take(f, tok, axis=0)
    ye = _grouped_ffn(blk_expert, fs, ew1, ew3, ew2, NBLK)
    y_sh = _grouped_ffn(jnp.zeros((NTB,), jnp.int32), f,
                        sw1[None], sw3[None], sw2[None], NTB)

    pos = dest.reshape(S, TOPK)
    y = (tw[:, 0:1] * jnp.take(ye, pos[:, 0], axis=0)
         + tw[:, 1:2] * jnp.take(ye, pos[:, 1], axis=0)
         + y_sh)
    return (h_att + y).reshape(b, s, DIM)


# speedup vs baseline: 1.1539x; 1.0992x over previous
import functools

import jax
import jax.numpy as jnp
from jax.experimental import pallas as pl
from jax.experimental.pallas import tpu as pltpu

DIM = 2048
NH = 16
DOWN = 512
UP = 2048
RHD = 32
VHD = 128
HID = 1408
NE = 8
TOPK = 2
EPS = 1e-5
S = 2048

QHD = UP // NH

BQ = 512
BK = 512
BM = 128
NHB = 2
BH = HID // NHB
P = S * TOPK + NE * BM
NBLK = P // BM
NTB = S // BM


def _rmsnorm(h, w):
    return h * jax.lax.rsqrt(jnp.mean(h * h, axis=-1, keepdims=True) + EPS) * w


def _rope(t, cs):
    t2 = t.reshape(t.shape[:-1] + (-1, 2))
    c = cs[:, None, :, 0]
    s = cs[:, None, :, 1]
    o0 = t2[..., 0] * c - t2[..., 1] * s
    o1 = t2[..., 0] * s + t2[..., 1] * c
    return jnp.stack([o0, o1], axis=-1).reshape(t.shape)


def _flash_kernel(qc_ref, qr_ref, kc_ref, kr_ref, v_ref, o_ref):
    qi = pl.program_id(1)
    scale = 1.0 / jnp.sqrt(jnp.float32(QHD + RHD))
    qc = qc_ref[...] * scale
    qr = qr_ref[0] * scale

    def scores(j):
        kc = kc_ref[pl.ds(j * BK, BK), :]
        kr = kr_ref[pl.ds(j * BK, BK), :]
        s = jax.lax.dot_general(qc, kc, (((1,), (1,)), ((), ())),
                                preferred_element_type=jnp.float32)
        s += jax.lax.dot_general(qr, kr, (((1,), (1,)), ((), ())),
                                 preferred_element_type=jnp.float32)
        return s

    def update(j, s, carry):
        acc, m, l = carry
        v = v_ref[pl.ds(j * BK, BK), :]
        m_new = jnp.maximum(m, jnp.max(s, axis=-1, keepdims=True))
        p = jnp.exp(s - m_new)
        alpha = jnp.exp(m - m_new)
        l_new = l * alpha + jnp.sum(p, axis=-1, keepdims=True)
        acc_new = acc * alpha + jax.lax.dot_general(
            p, v, (((1,), (0,)), ((), ())), preferred_element_type=jnp.float32)
        return acc_new, m_new, l_new

    def body(j, carry):
        return update(j, scores(j), carry)

    acc = jnp.zeros((BQ, VHD), jnp.float32)
    m0 = jnp.full((BQ, 1), -jnp.inf, jnp.float32)
    l0 = jnp.zeros((BQ, 1), jnp.float32)
    carry = jax.lax.fori_loop(0, qi, body, (acc, m0, l0))
    s = scores(qi)
    mask = (jax.lax.broadcasted_iota(jnp.int32, (BQ, BK), 0)
            >= jax.lax.broadcasted_iota(jnp.int32, (BQ, BK), 1))
    s = jnp.where(mask, s, -1e30)
    acc, m, l = update(qi, s, carry)
    o_ref[...] = acc / l


def _flash_attn(q_c, q_r, k_c, k_r, v):
    return pl.pallas_call(
        _flash_kernel,
        grid=(NH, S // BQ),
        in_specs=[
            pl.BlockSpec((BQ, QHD), lambda h, qi: (qi, h)),
            pl.BlockSpec((1, BQ, RHD), lambda h, qi: (h, qi, 0)),
            pl.BlockSpec((S, QHD), lambda h, qi: (0, h)),
            pl.BlockSpec((S, RHD), lambda h, qi: (0, 0)),
            pl.BlockSpec((S, VHD), lambda h, qi: (0, h)),
        ],
        out_specs=pl.BlockSpec((BQ, VHD), lambda h, qi: (qi, h)),
        out_shape=jax.ShapeDtypeStruct((S, NH * VHD), jnp.float32),
    )(q_c, q_r, k_c, k_r, v)


def _ffn_up_kernel(be_ref, fs_ref, w1_ref, w3_ref, g_ref):
    fs = fs_ref[...]
    h1 = jax.lax.dot_general(fs, w1_ref[0], (((1,), (1,)), ((), ())),
                             preferred_element_type=jnp.float32)
    h3 = jax.lax.dot_general(fs, w3_ref[0], (((1,), (1,)), ((), ())),
                             preferred_element_type=jnp.float32)
    g_ref[...] = h1 * jax.lax.logistic(h1) * h3


def _ffn_down_kernel(be_ref, g_ref, w2_ref, o_ref):
    o_ref[...] = jax.lax.dot_general(g_ref[...], w2_ref[0],
                                     (((1,), (1,)), ((), ())),
                                     preferred_element_type=jnp.float32)


def _grouped_ffn(be, fs, w1, w3, w2, nblk):
    g = pl.pallas_call(
        _ffn_up_kernel,
        grid_spec=pltpu.PrefetchScalarGridSpec(
            num_scalar_prefetch=1,
            grid=(nblk,),
            in_specs=[
                pl.BlockSpec((BM, DIM), lambda i, be: (i, 0)),
                pl.BlockSpec((1, HID, DIM), lambda i, be: (be[i], 0, 0)),
                pl.BlockSpec((1, HID, DIM), lambda i, be: (be[i], 0, 0)),
            ],
            out_specs=pl.BlockSpec((BM, HID), lambda i, be: (i, 0)),
        ),
        out_shape=jax.ShapeDtypeStruct((nblk * BM, HID), jnp.float32),
    )(be, fs, w1, w3)
    return pl.pallas_call(
        _ffn_down_kernel,
        grid_spec=pltpu.PrefetchScalarGridSpec(
            num_scalar_prefetch=1,
            grid=(nblk,),
            in_specs=[
                pl.BlockSpec((BM, HID), lambda i, be: (i, 0)),
                pl.BlockSpec((1, DIM, HID), lambda i, be: (be[i], 0, 0)),
            ],
            out_specs=pl.BlockSpec((BM, DIM), lambda i, be: (i, 0)),
        ),
        out_shape=jax.ShapeDtypeStruct((nblk * BM, DIM), jnp.float32),
    )(be, g, w2)


def _dot_t(a, w):
    return jax.lax.dot_general(a, w, (((1,), (1,)), ((), ())),
                               preferred_element_type=jnp.float32)


def _proj1_kernel(x_ref, nw_ref, wdd_ref, wkr_ref, bdd_ref, bkr_ref,
                  ckv_ref, cq_ref, kpe_ref):
    xb = x_ref[...]
    h = xb * jax.lax.rsqrt(jnp.mean(xb * xb, axis=-1, keepdims=True) + EPS)
    h = h * nw_ref[...]
    r = _dot_t(h, wdd_ref[...]) + bdd_ref[...]
    ckv_ref[...] = r[:, :DOWN]
    cq_ref[...] = r[:, DOWN:]
    kpe_ref[...] = _dot_t(h, wkr_ref[...]) + bkr_ref[...]


def _proj2_kernel(ckv_ref, cq_ref, wuk_ref, wuv_ref, wuq_ref, wqr_ref,
                  buk_ref, buv_ref, buq_ref, bqr_ref,
                  kc_ref, v_ref, qc_ref, qpe_ref):
    ckv = ckv_ref[...]
    cq = cq_ref[...]
    kc_ref[...] = _dot_t(ckv, wuk_ref[...]) + buk_ref[...]
    v_ref[...] = _dot_t(ckv, wuv_ref[...]) + buv_ref[...]
    qc_ref[...] = _dot_t(cq, wuq_ref[...]) + buq_ref[...]
    qpe_ref[...] = _dot_t(cq, wqr_ref[...]) + bqr_ref[...]


def _outproj_kernel(o_ref, x_ref, wo_ref, bo_ref, fnw_ref, gw_ref,
                    ha_ref, f_ref, lg_ref):
    ha = x_ref[...] + _dot_t(o_ref[...], wo_ref[...]) + bo_ref[...]
    ha_ref[...] = ha
    fb = ha * jax.lax.rsqrt(jnp.mean(ha * ha, axis=-1, keepdims=True) + EPS)
    fb = fb * fnw_ref[...]
    f_ref[...] = fb
    lg_ref[...] = _dot_t(fb, gw_ref[...])


def _row_spec(n):
    return pl.BlockSpec((1, n), lambda i: (0, 0))


def kernel(x, pos_cis, attn_norm_w, Wdkv, bdkv, Wuk, buk, Wuv, buv, Wdq, bdq,
           Wuq, buq, Wqr, bqr, Wkr, bkr, Wo, bo, ffn_norm_w, gate_w,
           ew1, ew2, ew3, sw1, sw2, sw3):
    b, s, _ = x.shape
    xf = x.reshape(s, DIM)

    BT1 = 512
    wdd = jnp.concatenate([Wdkv, Wdq], axis=0)
    bdd = jnp.concatenate([bdkv, bdq])[None]
    c_kv, c_q, k_pe = pl.pallas_call(
        _proj1_kernel,
        grid=(S // BT1,),
        in_specs=[
            pl.BlockSpec((BT1, DIM), lambda i: (i, 0)),
            _row_spec(DIM),
            pl.BlockSpec((2 * DOWN, DIM), lambda i: (0, 0)),
            pl.BlockSpec((RHD, DIM), lambda i: (0, 0)),
            _row_spec(2 * DOWN),
            _row_spec(RHD),
        ],
        out_specs=[
            pl.BlockSpec((BT1, DOWN), lambda i: (i, 0)),
            pl.BlockSpec((BT1, DOWN), lambda i: (i, 0)),
            pl.BlockSpec((BT1, RHD), lambda i: (i, 0)),
        ],
        out_shape=[
            jax.ShapeDtypeStruct((S, DOWN), jnp.float32),
            jax.ShapeDtypeStruct((S, DOWN), jnp.float32),
            jax.ShapeDtypeStruct((S, RHD), jnp.float32),
        ],
    )(xf, attn_norm_w[None], wdd, Wkr, bdd, bkr[None])

    BT2 = 256
    k_c, v, q_c, q_pe = pl.pallas_call(
        _proj2_kernel,
        grid=(S // BT2,),
        in_specs=[
            pl.BlockSpec((BT2, DOWN), lambda i: (i, 0)),
            pl.BlockSpec((BT2, DOWN), lambda i: (i, 0)),
            pl.BlockSpec((UP, DOWN), lambda i: (0, 0)),
            pl.BlockSpec((UP, DOWN), lambda i: (0, 0)),
            pl.BlockSpec((UP, DOWN), lambda i: (0, 0)),
            pl.BlockSpec((NH * RHD, DOWN), lambda i: (0, 0)),
            _row_spec(UP), _row_spec(UP), _row_spec(UP), _row_spec(NH * RHD),
        ],
        out_specs=[
            pl.BlockSpec((BT2, UP), lambda i: (i, 0)),
            pl.BlockSpec((BT2, UP), lambda i: (i, 0)),
            pl.BlockSpec((BT2, UP), lambda i: (i, 0)),
            pl.BlockSpec((BT2, NH * RHD), lambda i: (i, 0)),
        ],
        out_shape=[
            jax.ShapeDtypeStruct((S, UP), jnp.float32),
            jax.ShapeDtypeStruct((S, UP), jnp.float32),
            jax.ShapeDtypeStruct((S, UP), jnp.float32),
            jax.ShapeDtypeStruct((S, NH * RHD), jnp.float32),
        ],
    )(c_kv, c_q, Wuk, Wuv, Wuq, Wqr, buk[None], buv[None], buq[None],
      bqr[None])

    q_r = _rope(q_pe.reshape(s, NH, RHD), pos_cis).transpose(1, 0, 2)
    k_r = _rope(k_pe.reshape(s, 1, RHD), pos_cis).reshape(s, RHD)

    o = _flash_attn(q_c, q_r, k_c, k_r, v)

    BT3 = 256
    h_att, f, logits = pl.pallas_call(
        _outproj_kernel,
        grid=(S // BT3,),
        in_specs=[
            pl.BlockSpec((BT3, DIM), lambda i: (i, 0)),
            pl.BlockSpec((BT3, DIM), lambda i: (i, 0)),
            pl.BlockSpec((DIM, NH * VHD), lambda i: (0, 0)),
            _row_spec(DIM),
            _row_spec(DIM),
            pl.BlockSpec((NE, DIM), lambda i: (0, 0)),
        ],
        out_specs=[
            pl.BlockSpec((BT3, DIM), lambda i: (i, 0)),
            pl.BlockSpec((BT3, DIM), lambda i: (i, 0)),
            pl.BlockSpec((BT3, NE), lambda i: (i, 0)),
        ],
        out_shape=[
            jax.ShapeDtypeStruct((S, DIM), jnp.float32),
            jax.ShapeDtypeStruct((S, DIM), jnp.float32),
            jax.ShapeDtypeStruct((S, NE), jnp.float32),
        ],
    )(o, xf, Wo, bo[None], ffn_norm_w[None], gate_w)

    sc = jax.nn.softmax(logits, axis=-1)
    lanes = jnp.arange(NE, dtype=jnp.int32)[None, :]
    i1 = jnp.argmax(sc, axis=-1).astype(jnp.int32)
    w1v = jnp.max(sc, axis=-1)
    sc2 = jnp.where(lanes == i1[:, None], -jnp.inf, sc)
    i2 = jnp.argmax(sc2, axis=-1).astype(jnp.int32)
    w2v = jnp.max(sc2, axis=-1)
    tw = jnp.stack([w1v, w2v], axis=-1)
    ti = jnp.stack([i1, i2], axis=-1)

    e_flat = ti.reshape(-1)
    oh = (e_flat[:, None] == lanes).astype(jnp.int32)
    ranks = jnp.cumsum(oh, axis=0) - oh
    rank = jnp.take_along_axis(ranks, e_flat[:, None], axis=1)[:, 0]
    counts = ranks[-1] + oh[-1]
    pc = ((counts + BM - 1) // BM) * BM
    padded_start = jnp.cumsum(pc) - pc
    dest = padded_start[e_flat] + rank
    src = jnp.zeros((P,), jnp.int32).at[dest].set(
        jnp.arange(S * TOPK, dtype=jnp.int32))
    tok = src // TOPK
    blk_start = padded_start // BM
    bids = jnp.arange(NBLK, dtype=jnp.int32)
    blk_expert = (jnp.sum(bids[:, None] >= blk_start[None, :], axis=1)
                  .astype(jnp.int32) - 1)

    fs = jnp.take(f, tok, axis=0)
    ye = _grouped_ffn(blk_expert, fs, ew1, ew3, ew2, NBLK)
    y_sh = _grouped_ffn(jnp.zeros((NTB,), jnp.int32), f,
                        sw1[None], sw3[None], sw2[None], NTB)

    pos = dest.reshape(S, TOPK)
    y = (tw[:, 0:1] * jnp.take(ye, pos[:, 0], axis=0)
         + tw[:, 1:2] * jnp.take(ye, pos[:, 1], axis=0)
         + y_sh)
    return (h_att + y).reshape(b, s, DIM)


# speedup vs baseline: 1.1643x; 1.0090x over previous
import functools

import jax
import jax.numpy as jnp
from jax.experimental import pallas as pl
from jax.experimental.pallas import tpu as pltpu

DIM = 2048
NH = 16
DOWN = 512
UP = 2048
RHD = 32
VHD = 128
HID = 1408
NE = 8
TOPK = 2
EPS = 1e-5
S = 2048

QHD = UP // NH

BQ = 512
BK = 512
BM = 128
NHB = 2
BH = HID // NHB
P = S * TOPK + NE * BM
NBLK = P // BM
NTB = S // BM


def _rmsnorm(h, w):
    return h * jax.lax.rsqrt(jnp.mean(h * h, axis=-1, keepdims=True) + EPS) * w


def _rope(t, cs):
    t2 = t.reshape(t.shape[:-1] + (-1, 2))
    c = cs[:, None, :, 0]
    s = cs[:, None, :, 1]
    o0 = t2[..., 0] * c - t2[..., 1] * s
    o1 = t2[..., 0] * s + t2[..., 1] * c
    return jnp.stack([o0, o1], axis=-1).reshape(t.shape)


def _flash_kernel(qc_ref, qr_ref, kc_ref, kr_ref, v_ref, o_ref):
    qi = pl.program_id(1)
    scale = 1.0 / jnp.sqrt(jnp.float32(QHD + RHD))
    qc = qc_ref[...] * scale
    qr = qr_ref[0] * scale

    qcb = qc.astype(jnp.bfloat16)
    qrb = qr.astype(jnp.bfloat16)

    def scores(j):
        kc = kc_ref[pl.ds(j * BK, BK), :].astype(jnp.bfloat16)
        kr = kr_ref[pl.ds(j * BK, BK), :].astype(jnp.bfloat16)
        s = jax.lax.dot_general(qcb, kc, (((1,), (1,)), ((), ())),
                                preferred_element_type=jnp.float32)
        s += jax.lax.dot_general(qrb, kr, (((1,), (1,)), ((), ())),
                                 preferred_element_type=jnp.float32)
        return s

    def update(j, s, carry):
        acc, m, l = carry
        v = v_ref[pl.ds(j * BK, BK), :].astype(jnp.bfloat16)
        m_new = jnp.maximum(m, jnp.max(s, axis=-1, keepdims=True))
        p = jnp.exp(s - m_new)
        alpha = jnp.exp(m - m_new)
        l_new = l * alpha + jnp.sum(p, axis=-1, keepdims=True)
        acc_new = acc * alpha + jax.lax.dot_general(
            p.astype(jnp.bfloat16), v, (((1,), (0,)), ((), ())),
            preferred_element_type=jnp.float32)
        return acc_new, m_new, l_new

    def body(j, carry):
        return update(j, scores(j), carry)

    acc = jnp.zeros((BQ, VHD), jnp.float32)
    m0 = jnp.full((BQ, 1), -jnp.inf, jnp.float32)
    l0 = jnp.zeros((BQ, 1), jnp.float32)
    carry = jax.lax.fori_loop(0, qi, body, (acc, m0, l0))
    s = scores(qi)
    mask = (jax.lax.broadcasted_iota(jnp.int32, (BQ, BK), 0)
            >= jax.lax.broadcasted_iota(jnp.int32, (BQ, BK), 1))
    s = jnp.where(mask, s, -1e30)
    acc, m, l = update(qi, s, carry)
    o_ref[...] = acc / l


def _flash_attn(q_c, q_r, k_c, k_r, v):
    return pl.pallas_call(
        _flash_kernel,
        grid=(NH, S // BQ),
        in_specs=[
            pl.BlockSpec((BQ, QHD), lambda h, qi: (qi, h)),
            pl.BlockSpec((1, BQ, RHD), lambda h, qi: (h, qi, 0)),
            pl.BlockSpec((S, QHD), lambda h, qi: (0, h)),
            pl.BlockSpec((S, RHD), lambda h, qi: (0, 0)),
            pl.BlockSpec((S, VHD), lambda h, qi: (0, h)),
        ],
        out_specs=pl.BlockSpec((BQ, VHD), lambda h, qi: (qi, h)),
        out_shape=jax.ShapeDtypeStruct((S, NH * VHD), jnp.float32),
    )(q_c, q_r, k_c, k_r, v)


def _ffn_up_kernel(be_ref, fs_ref, w1_ref, w3_ref, g_ref):
    fs = fs_ref[...]
    h1 = jax.lax.dot_general(fs, w1_ref[0], (((1,), (1,)), ((), ())),
                             preferred_element_type=jnp.float32)
    h3 = jax.lax.dot_general(fs, w3_ref[0], (((1,), (1,)), ((), ())),
                             preferred_element_type=jnp.float32)
    g_ref[...] = h1 * jax.lax.logistic(h1) * h3


def _ffn_down_kernel(be_ref, g_ref, w2_ref, o_ref):
    o_ref[...] = jax.lax.dot_general(g_ref[...], w2_ref[0],
                                     (((1,), (1,)), ((), ())),
                                     preferred_element_type=jnp.float32)


def _grouped_ffn(be, fs, w1, w3, w2, nblk):
    g = pl.pallas_call(
        _ffn_up_kernel,
        grid_spec=pltpu.PrefetchScalarGridSpec(
            num_scalar_prefetch=1,
            grid=(nblk,),
            in_specs=[
                pl.BlockSpec((BM, DIM), lambda i, be: (i, 0)),
                pl.BlockSpec((1, HID, DIM), lambda i, be: (be[i], 0, 0)),
                pl.BlockSpec((1, HID, DIM), lambda i, be: (be[i], 0, 0)),
            ],
            out_specs=pl.BlockSpec((BM, HID), lambda i, be: (i, 0)),
        ),
        out_shape=jax.ShapeDtypeStruct((nblk * BM, HID), jnp.float32),
    )(be, fs, w1, w3)
    return pl.pallas_call(
        _ffn_down_kernel,
        grid_spec=pltpu.PrefetchScalarGridSpec(
            num_scalar_prefetch=1,
            grid=(nblk,),
            in_specs=[
                pl.BlockSpec((BM, HID), lambda i, be: (i, 0)),
                pl.BlockSpec((1, DIM, HID), lambda i, be: (be[i], 0, 0)),
            ],
            out_specs=pl.BlockSpec((BM, DIM), lambda i, be: (i, 0)),
        ),
        out_shape=jax.ShapeDtypeStruct((nblk * BM, DIM), jnp.float32),
    )(be, g, w2)


def _dot_t(a, w):
    return jax.lax.dot_general(a, w, (((1,), (1,)), ((), ())),
                               preferred_element_type=jnp.float32)


def _proj1_kernel(x_ref, nw_ref, wdd_ref, wkr_ref, bdd_ref, bkr_ref,
                  ckv_ref, cq_ref, kpe_ref):
    xb = x_ref[...]
    h = xb * jax.lax.rsqrt(jnp.mean(xb * xb, axis=-1, keepdims=True) + EPS)
    h = h * nw_ref[...]
    r = _dot_t(h, wdd_ref[...]) + bdd_ref[...]
    ckv_ref[...] = r[:, :DOWN]
    cq_ref[...] = r[:, DOWN:]
    kpe_ref[...] = _dot_t(h, wkr_ref[...]) + bkr_ref[...]


def _proj2_kernel(ckv_ref, cq_ref, wuk_ref, wuv_ref, wuq_ref, wqr_ref,
                  buk_ref, buv_ref, buq_ref, bqr_ref,
                  kc_ref, v_ref, qc_ref, qpe_ref):
    ckv = ckv_ref[...]
    cq = cq_ref[...]
    kc_ref[...] = _dot_t(ckv, wuk_ref[...]) + buk_ref[...]
    v_ref[...] = _dot_t(ckv, wuv_ref[...]) + buv_ref[...]
    qc_ref[...] = _dot_t(cq, wuq_ref[...]) + buq_ref[...]
    qpe_ref[...] = _dot_t(cq, wqr_ref[...]) + bqr_ref[...]


def _outproj_kernel(o_ref, x_ref, wo_ref, bo_ref, fnw_ref, gw_ref,
                    ha_ref, f_ref, lg_ref):
    ha = x_ref[...] + _dot_t(o_ref[...], wo_ref[...]) + bo_ref[...]
    ha_ref[...] = ha
    fb = ha * jax.lax.rsqrt(jnp.mean(ha * ha, axis=-1, keepdims=True) + EPS)
    fb = fb * fnw_ref[...]
    f_ref[...] = fb
    lg_ref[...] = _dot_t(fb, gw_ref[...])


def _row_spec(n):
    return pl.BlockSpec((1, n), lambda i: (0, 0))


def kernel(x, pos_cis, attn_norm_w, Wdkv, bdkv, Wuk, buk, Wuv, buv, Wdq, bdq,
           Wuq, buq, Wqr, bqr, Wkr, bkr, Wo, bo, ffn_norm_w, gate_w,
           ew1, ew2, ew3, sw1, sw2, sw3):
    b, s, _ = x.shape
    xf = x.reshape(s, DIM)

    BT1 = 512
    wdd = jnp.concatenate([Wdkv, Wdq], axis=0)
    bdd = jnp.concatenate([bdkv, bdq])[None]
    c_kv, c_q, k_pe = pl.pallas_call(
        _proj1_kernel,
        grid=(S // BT1,),
        in_specs=[
            pl.BlockSpec((BT1, DIM), lambda i: (i, 0)),
            _row_spec(DIM),
            pl.BlockSpec((2 * DOWN, DIM), lambda i: (0, 0)),
            pl.BlockSpec((RHD, DIM), lambda i: (0, 0)),
            _row_spec(2 * DOWN),
            _row_spec(RHD),
        ],
        out_specs=[
            pl.BlockSpec((BT1, DOWN), lambda i: (i, 0)),
            pl.BlockSpec((BT1, DOWN), lambda i: (i, 0)),
            pl.BlockSpec((BT1, RHD), lambda i: (i, 0)),
        ],
        out_shape=[
            jax.ShapeDtypeStruct((S, DOWN), jnp.float32),
            jax.ShapeDtypeStruct((S, DOWN), jnp.float32),
            jax.ShapeDtypeStruct((S, RHD), jnp.float32),
        ],
    )(xf, attn_norm_w[None], wdd, Wkr, bdd, bkr[None])

    BT2 = 256
    k_c, v, q_c, q_pe = pl.pallas_call(
        _proj2_kernel,
        grid=(S // BT2,),
        in_specs=[
            pl.BlockSpec((BT2, DOWN), lambda i: (i, 0)),
            pl.BlockSpec((BT2, DOWN), lambda i: (i, 0)),
            pl.BlockSpec((UP, DOWN), lambda i: (0, 0)),
            pl.BlockSpec((UP, DOWN), lambda i: (0, 0)),
            pl.BlockSpec((UP, DOWN), lambda i: (0, 0)),
            pl.BlockSpec((NH * RHD, DOWN), lambda i: (0, 0)),
            _row_spec(UP), _row_spec(UP), _row_spec(UP), _row_spec(NH * RHD),
        ],
        out_specs=[
            pl.BlockSpec((BT2, UP), lambda i: (i, 0)),
            pl.BlockSpec((BT2, UP), lambda i: (i, 0)),
            pl.BlockSpec((BT2, UP), lambda i: (i, 0)),
            pl.BlockSpec((BT2, NH * RHD), lambda i: (i, 0)),
        ],
        out_shape=[
            jax.ShapeDtypeStruct((S, UP), jnp.float32),
            jax.ShapeDtypeStruct((S, UP), jnp.float32),
            jax.ShapeDtypeStruct((S, UP), jnp.float32),
            jax.ShapeDtypeStruct((S, NH * RHD), jnp.float32),
        ],
    )(c_kv, c_q, Wuk, Wuv, Wuq, Wqr, buk[None], buv[None], buq[None],
      bqr[None])

    q_r = _rope(q_pe.reshape(s, NH, RHD), pos_cis).transpose(1, 0, 2)
    k_r = _rope(k_pe.reshape(s, 1, RHD), pos_cis).reshape(s, RHD)

    o = _flash_attn(q_c, q_r, k_c, k_r, v)

    BT3 = 256
    h_att, f, logits = pl.pallas_call(
        _outproj_kernel,
        grid=(S // BT3,),
        in_specs=[
            pl.BlockSpec((BT3, DIM), lambda i: (i, 0)),
            pl.BlockSpec((BT3, DIM), lambda i: (i, 0)),
            pl.BlockSpec((DIM, NH * VHD), lambda i: (0, 0)),
            _row_spec(DIM),
            _row_spec(DIM),
            pl.BlockSpec((NE, DIM), lambda i: (0, 0)),
        ],
        out_specs=[
            pl.BlockSpec((BT3, DIM), lambda i: (i, 0)),
            pl.BlockSpec((BT3, DIM), lambda i: (i, 0)),
            pl.BlockSpec((BT3, NE), lambda i: (i, 0)),
        ],
        out_shape=[
            jax.ShapeDtypeStruct((S, DIM), jnp.float32),
            jax.ShapeDtypeStruct((S, DIM), jnp.float32),
            jax.ShapeDtypeStruct((S, NE), jnp.float32),
        ],
    )(o, xf, Wo, bo[None], ffn_norm_w[None], gate_w)

    sc = jax.nn.softmax(logits, axis=-1)
    lanes = jnp.arange(NE, dtype=jnp.int32)[None, :]
    i1 = jnp.argmax(sc, axis=-1).astype(jnp.int32)
    w1v = jnp.max(sc, axis=-1)
    sc2 = jnp.where(lanes == i1[:, None], -jnp.inf, sc)
    i2 = jnp.argmax(sc2, axis=-1).astype(jnp.int32)
    w2v = jnp.max(sc2, axis=-1)
    tw = jnp.stack([w1v, w2v], axis=-1)
    ti = jnp.stack([i1, i2], axis=-1)

    e_flat = ti.reshape(-1)
    oh = (e_flat[:, None] == lanes).astype(jnp.float32)
    NC = 128
    ohc = oh.reshape((S * TOPK) // NC, NC, NE)
    ar = jnp.arange(NC)
    trils = (ar[:, None] > ar[None, :]).astype(jnp.float32)
    within = jnp.einsum('rc,bcn->brn', trils, ohc)
    chunk_tot = ohc.sum(axis=1)
    nch = (S * TOPK) // NC
    arc = jnp.arange(nch)
    trils_c = (arc[:, None] > arc[None, :]).astype(jnp.float32)
    chunk_pref = trils_c @ chunk_tot
    ranks = (within + chunk_pref[:, None, :]).reshape(S * TOPK, NE)
    rank = jnp.sum(ranks * oh, axis=1).astype(jnp.int32)
    counts = chunk_tot.sum(axis=0).astype(jnp.int32)
    pc = ((counts + BM - 1) // BM) * BM
    padded_start = jnp.cumsum(pc) - pc
    dest = padded_start[e_flat] + rank
    src = jnp.zeros((P,), jnp.int32).at[dest].set(
        jnp.arange(S * TOPK, dtype=jnp.int32))
    tok = src // TOPK
    blk_start = padded_start // BM
    bids = jnp.arange(NBLK, dtype=jnp.int32)
    blk_expert = (jnp.sum(bids[:, None] >= blk_start[None, :], axis=1)
                  .astype(jnp.int32) - 1)

    fs = jnp.take(f, tok, axis=0)
    ye = _grouped_ffn(blk_expert, fs, ew1, ew3, ew2, NBLK)
    y_sh = _grouped_ffn(jnp.zeros((NTB,), jnp.int32), f,
                        sw1[None], sw3[None], sw2[None], NTB)

    pos = dest.reshape(S, TOPK)
    y = (tw[:, 0:1] * jnp.take(ye, pos[:, 0], axis=0)
         + tw[:, 1:2] * jnp.take(ye, pos[:, 1], axis=0)
         + y_sh)
    return (h_att + y).reshape(b, s, DIM)


# speedup vs baseline: 1.1650x; 1.0006x over previous
import functools

import jax
import jax.numpy as jnp
from jax.experimental import pallas as pl
from jax.experimental.pallas import tpu as pltpu

DIM = 2048
NH = 16
DOWN = 512
UP = 2048
RHD = 32
VHD = 128
HID = 1408
NE = 8
TOPK = 2
EPS = 1e-5
S = 2048

QHD = UP // NH

BQ = 512
BK = 512
BM = 128
NHB = 2
BH = HID // NHB
P = S * TOPK + NE * BM
NBLK = P // BM
NTB = S // BM


def _rmsnorm(h, w):
    return h * jax.lax.rsqrt(jnp.mean(h * h, axis=-1, keepdims=True) + EPS) * w


def _rope(t, cs):
    t2 = t.reshape(t.shape[:-1] + (-1, 2))
    c = cs[:, None, :, 0]
    s = cs[:, None, :, 1]
    o0 = t2[..., 0] * c - t2[..., 1] * s
    o1 = t2[..., 0] * s + t2[..., 1] * c
    return jnp.stack([o0, o1], axis=-1).reshape(t.shape)


def _flash_kernel(qc_ref, qr_ref, kc_ref, kr_ref, v_ref, o_ref):
    qi = pl.program_id(1)
    scale = 1.0 / jnp.sqrt(jnp.float32(QHD + RHD))
    qc = qc_ref[...] * scale
    qr = qr_ref[0] * scale

    qcb = qc.astype(jnp.bfloat16)
    qrb = qr.astype(jnp.bfloat16)

    def scores(j):
        kc = kc_ref[pl.ds(j * BK, BK), :].astype(jnp.bfloat16)
        kr = kr_ref[pl.ds(j * BK, BK), :].astype(jnp.bfloat16)
        s = jax.lax.dot_general(qcb, kc, (((1,), (1,)), ((), ())),
                                preferred_element_type=jnp.float32)
        s += jax.lax.dot_general(qrb, kr, (((1,), (1,)), ((), ())),
                                 preferred_element_type=jnp.float32)
        return s

    def update(j, s, carry):
        acc, m, l = carry
        v = v_ref[pl.ds(j * BK, BK), :].astype(jnp.bfloat16)
        m_new = jnp.maximum(m, jnp.max(s, axis=-1, keepdims=True))
        p = jnp.exp(s - m_new)
        alpha = jnp.exp(m - m_new)
        l_new = l * alpha + jnp.sum(p, axis=-1, keepdims=True)
        acc_new = acc * alpha + jax.lax.dot_general(
            p.astype(jnp.bfloat16), v, (((1,), (0,)), ((), ())),
            preferred_element_type=jnp.float32)
        return acc_new, m_new, l_new

    def body(j, carry):
        return update(j, scores(j), carry)

    acc = jnp.zeros((BQ, VHD), jnp.float32)
    m0 = jnp.full((BQ, 1), -jnp.inf, jnp.float32)
    l0 = jnp.zeros((BQ, 1), jnp.float32)
    carry = jax.lax.fori_loop(0, qi, body, (acc, m0, l0))
    s = scores(qi)
    mask = (jax.lax.broadcasted_iota(jnp.int32, (BQ, BK), 0)
            >= jax.lax.broadcasted_iota(jnp.int32, (BQ, BK), 1))
    s = jnp.where(mask, s, -1e30)
    acc, m, l = update(qi, s, carry)
    o_ref[...] = acc / l


def _flash_attn(q_c, q_r, k_c, k_r, v):
    return pl.pallas_call(
        _flash_kernel,
        grid=(NH, S // BQ),
        in_specs=[
            pl.BlockSpec((BQ, QHD), lambda h, qi: (qi, h)),
            pl.BlockSpec((1, BQ, RHD), lambda h, qi: (h, qi, 0)),
            pl.BlockSpec((S, QHD), lambda h, qi: (0, h)),
            pl.BlockSpec((S, RHD), lambda h, qi: (0, 0)),
            pl.BlockSpec((S, VHD), lambda h, qi: (0, h)),
        ],
        out_specs=pl.BlockSpec((BQ, VHD), lambda h, qi: (qi, h)),
        out_shape=jax.ShapeDtypeStruct((S, NH * VHD), jnp.float32),
    )(q_c, q_r, k_c, k_r, v)


def _ffn_up_kernel(be_ref, fs_ref, w1_ref, w3_ref, g_ref):
    fs = fs_ref[...]
    h1 = jax.lax.dot_general(fs, w1_ref[0], (((1,), (1,)), ((), ())),
                             preferred_element_type=jnp.float32)
    h3 = jax.lax.dot_general(fs, w3_ref[0], (((1,), (1,)), ((), ())),
                             preferred_element_type=jnp.float32)
    g_ref[...] = h1 * jax.lax.logistic(h1) * h3


def _ffn_down_kernel(be_ref, g_ref, w2_ref, o_ref):
    o_ref[...] = jax.lax.dot_general(g_ref[...], w2_ref[0],
                                     (((1,), (1,)), ((), ())),
                                     preferred_element_type=jnp.float32)


def _grouped_ffn(be, fs, w1, w3, w2, nblk):
    g = pl.pallas_call(
        _ffn_up_kernel,
        grid_spec=pltpu.PrefetchScalarGridSpec(
            num_scalar_prefetch=1,
            grid=(nblk,),
            in_specs=[
                pl.BlockSpec((BM, DIM), lambda i, be: (i, 0)),
                pl.BlockSpec((1, HID, DIM), lambda i, be: (be[i], 0, 0)),
                pl.BlockSpec((1, HID, DIM), lambda i, be: (be[i], 0, 0)),
            ],
            out_specs=pl.BlockSpec((BM, HID), lambda i, be: (i, 0)),
        ),
        out_shape=jax.ShapeDtypeStruct((nblk * BM, HID), jnp.float32),
    )(be, fs, w1, w3)
    return pl.pallas_call(
        _ffn_down_kernel,
        grid_spec=pltpu.PrefetchScalarGridSpec(
            num_scalar_prefetch=1,
            grid=(nblk,),
            in_specs=[
                pl.BlockSpec((BM, HID), lambda i, be: (i, 0)),
                pl.BlockSpec((1, DIM, HID), lambda i, be: (be[i], 0, 0)),
            ],
            out_specs=pl.BlockSpec((BM, DIM), lambda i, be: (i, 0)),
        ),
        out_shape=jax.ShapeDtypeStruct((nblk * BM, DIM), jnp.float32),
    )(be, g, w2)


def _dot_t(a, w):
    return jax.lax.dot_general(a, w, (((1,), (1,)), ((), ())),
                               preferred_element_type=jnp.float32)


def _dot_tb(a, w):
    return jax.lax.dot_general(a.astype(jnp.bfloat16), w.astype(jnp.bfloat16),
                               (((1,), (1,)), ((), ())),
                               preferred_element_type=jnp.float32)


def _proj1_kernel(x_ref, nw_ref, wdd_ref, wkr_ref, bdd_ref, bkr_ref,
                  ckv_ref, cq_ref, kpe_ref):
    xb = x_ref[...]
    h = xb * jax.lax.rsqrt(jnp.mean(xb * xb, axis=-1, keepdims=True) + EPS)
    h = h * nw_ref[...]
    r = _dot_tb(h, wdd_ref[...]) + bdd_ref[...]
    ckv_ref[...] = r[:, :DOWN]
    cq_ref[...] = r[:, DOWN:]
    kpe_ref[...] = _dot_tb(h, wkr_ref[...]) + bkr_ref[...]


def _proj2_kernel(ckv_ref, cq_ref, wuk_ref, wuv_ref, wuq_ref, wqr_ref,
                  buk_ref, buv_ref, buq_ref, bqr_ref,
                  kc_ref, v_ref, qc_ref, qpe_ref):
    ckv = ckv_ref[...]
    cq = cq_ref[...]
    kc_ref[...] = _dot_tb(ckv, wuk_ref[...]) + buk_ref[...]
    v_ref[...] = _dot_tb(ckv, wuv_ref[...]) + buv_ref[...]
    qc_ref[...] = _dot_tb(cq, wuq_ref[...]) + buq_ref[...]
    qpe_ref[...] = _dot_tb(cq, wqr_ref[...]) + bqr_ref[...]


def _outproj_kernel(o_ref, x_ref, wo_ref, bo_ref, fnw_ref, gw_ref,
                    ha_ref, f_ref, lg_ref):
    ha = x_ref[...] + _dot_tb(o_ref[...], wo_ref[...]) + bo_ref[...]
    ha_ref[...] = ha
    fb = ha * jax.lax.rsqrt(jnp.mean(ha * ha, axis=-1, keepdims=True) + EPS)
    fb = fb * fnw_ref[...]
    f_ref[...] = fb
    lg_ref[...] = _dot_t(fb, gw_ref[...])


def _row_spec(n):
    return pl.BlockSpec((1, n), lambda i: (0, 0))


def kernel(x, pos_cis, attn_norm_w, Wdkv, bdkv, Wuk, buk, Wuv, buv, Wdq, bdq,
           Wuq, buq, Wqr, bqr, Wkr, bkr, Wo, bo, ffn_norm_w, gate_w,
           ew1, ew2, ew3, sw1, sw2, sw3):
    b, s, _ = x.shape
    xf = x.reshape(s, DIM)

    BT1 = 512
    wdd = jnp.concatenate([Wdkv, Wdq], axis=0)
    bdd = jnp.concatenate([bdkv, bdq])[None]
    c_kv, c_q, k_pe = pl.pallas_call(
        _proj1_kernel,
        grid=(S // BT1,),
        in_specs=[
            pl.BlockSpec((BT1, DIM), lambda i: (i, 0)),
            _row_spec(DIM),
            pl.BlockSpec((2 * DOWN, DIM), lambda i: (0, 0)),
            pl.BlockSpec((RHD, DIM), lambda i: (0, 0)),
            _row_spec(2 * DOWN),
            _row_spec(RHD),
        ],
        out_specs=[
            pl.BlockSpec((BT1, DOWN), lambda i: (i, 0)),
            pl.BlockSpec((BT1, DOWN), lambda i: (i, 0)),
            pl.BlockSpec((BT1, RHD), lambda i: (i, 0)),
        ],
        out_shape=[
            jax.ShapeDtypeStruct((S, DOWN), jnp.float32),
            jax.ShapeDtypeStruct((S, DOWN), jnp.float32),
            jax.ShapeDtypeStruct((S, RHD), jnp.float32),
        ],
    )(xf, attn_norm_w[None], wdd, Wkr, bdd, bkr[None])

    BT2 = 256
    k_c, v, q_c, q_pe = pl.pallas_call(
        _proj2_kernel,
        grid=(S // BT2,),
        in_specs=[
            pl.BlockSpec((BT2, DOWN), lambda i: (i, 0)),
            pl.BlockSpec((BT2, DOWN), lambda i: (i, 0)),
            pl.BlockSpec((UP, DOWN), lambda i: (0, 0)),
            pl.BlockSpec((UP, DOWN), lambda i: (0, 0)),
            pl.BlockSpec((UP, DOWN), lambda i: (0, 0)),
            pl.BlockSpec((NH * RHD, DOWN), lambda i: (0, 0)),
            _row_spec(UP), _row_spec(UP), _row_spec(UP), _row_spec(NH * RHD),
        ],
        out_specs=[
            pl.BlockSpec((BT2, UP), lambda i: (i, 0)),
            pl.BlockSpec((BT2, UP), lambda i: (i, 0)),
            pl.BlockSpec((BT2, UP), lambda i: (i, 0)),
            pl.BlockSpec((BT2, NH * RHD), lambda i: (i, 0)),
        ],
        out_shape=[
            jax.ShapeDtypeStruct((S, UP), jnp.float32),
            jax.ShapeDtypeStruct((S, UP), jnp.float32),
            jax.ShapeDtypeStruct((S, UP), jnp.float32),
            jax.ShapeDtypeStruct((S, NH * RHD), jnp.float32),
        ],
    )(c_kv, c_q, Wuk, Wuv, Wuq, Wqr, buk[None], buv[None], buq[None],
      bqr[None])

    q_r = _rope(q_pe.reshape(s, NH, RHD), pos_cis).transpose(1, 0, 2)
    k_r = _rope(k_pe.reshape(s, 1, RHD), pos_cis).reshape(s, RHD)

    o = _flash_attn(q_c, q_r, k_c, k_r, v)

    BT3 = 256
    h_att, f, logits = pl.pallas_call(
        _outproj_kernel,
        grid=(S // BT3,),
        in_specs=[
            pl.BlockSpec((BT3, DIM), lambda i: (i, 0)),
            pl.BlockSpec((BT3, DIM), lambda i: (i, 0)),
            pl.BlockSpec((DIM, NH * VHD), lambda i: (0, 0)),
            _row_spec(DIM),
            _row_spec(DIM),
            pl.BlockSpec((NE, DIM), lambda i: (0, 0)),
        ],
        out_specs=[
            pl.BlockSpec((BT3, DIM), lambda i: (i, 0)),
            pl.BlockSpec((BT3, DIM), lambda i: (i, 0)),
            pl.BlockSpec((BT3, NE), lambda i: (i, 0)),
        ],
        out_shape=[
            jax.ShapeDtypeStruct((S, DIM), jnp.float32),
            jax.ShapeDtypeStruct((S, DIM), jnp.float32),
            jax.ShapeDtypeStruct((S, NE), jnp.float32),
        ],
    )(o, xf, Wo, bo[None], ffn_norm_w[None], gate_w)

    sc = jax.nn.softmax(logits, axis=-1)
    lanes = jnp.arange(NE, dtype=jnp.int32)[None, :]
    i1 = jnp.argmax(sc, axis=-1).astype(jnp.int32)
    w1v = jnp.max(sc, axis=-1)
    sc2 = jnp.where(lanes == i1[:, None], -jnp.inf, sc)
    i2 = jnp.argmax(sc2, axis=-1).astype(jnp.int32)
    w2v = jnp.max(sc2, axis=-1)
    tw = jnp.stack([w1v, w2v], axis=-1)
    ti = jnp.stack([i1, i2], axis=-1)

    e_flat = ti.reshape(-1)
    oh = (e_flat[:, None] == lanes).astype(jnp.float32)
    NC = 128
    ohc = oh.reshape((S * TOPK) // NC, NC, NE)
    ar = jnp.arange(NC)
    trils = (ar[:, None] > ar[None, :]).astype(jnp.float32)
    within = jnp.einsum('rc,bcn->brn', trils, ohc)
    chunk_tot = ohc.sum(axis=1)
    nch = (S * TOPK) // NC
    arc = jnp.arange(nch)
    trils_c = (arc[:, None] > arc[None, :]).astype(jnp.float32)
    chunk_pref = trils_c @ chunk_tot
    ranks = (within + chunk_pref[:, None, :]).reshape(S * TOPK, NE)
    rank = jnp.sum(ranks * oh, axis=1).astype(jnp.int32)
    counts = chunk_tot.sum(axis=0).astype(jnp.int32)
    pc = ((counts + BM - 1) // BM) * BM
    padded_start = jnp.cumsum(pc) - pc
    dest = padded_start[e_flat] + rank
    src = jnp.zeros((P,), jnp.int32).at[dest].set(
        jnp.arange(S * TOPK, dtype=jnp.int32))
    tok = src // TOPK
    blk_start = padded_start // BM
    bids = jnp.arange(NBLK, dtype=jnp.int32)
    blk_expert = (jnp.sum(bids[:, None] >= blk_start[None, :], axis=1)
                  .astype(jnp.int32) - 1)

    fs = jnp.take(f, tok, axis=0)
    ye = _grouped_ffn(blk_expert, fs, ew1, ew3, ew2, NBLK)
    y_sh = _grouped_ffn(jnp.zeros((NTB,), jnp.int32), f,
                        sw1[None], sw3[None], sw2[None], NTB)

    pos = dest.reshape(S, TOPK)
    y = (tw[:, 0:1] * jnp.take(ye, pos[:, 0], axis=0)
         + tw[:, 1:2] * jnp.take(ye, pos[:, 1], axis=0)
         + y_sh)
    return (h_att + y).reshape(b, s, DIM)
